# baseline (speedup 1.0000x reference)
import functools

import jax
import jax.numpy as jnp
from jax import lax
from jax.experimental import pallas as pl
from jax.experimental.pallas import tpu as pltpu
from jax.experimental.pallas import tpu_sc as plsc

_TB = 1024
_TBF = 512
_SC_CHUNKS = (8192, 8192, 8192)
_TC_TAIL = 8192


def _affinity_block(x_ref, w_ref, aff_ref):
    x = x_ref[...]
    logits = jnp.dot(x, w_ref[...], preferred_element_type=jnp.float32)
    sp = jnp.maximum(logits, 0.0) + jnp.log1p(jnp.exp(-jnp.abs(logits)))
    aff_ref[...] = jnp.sqrt(sp + 1e-12)


def _affinity_call(x, W, start, size):
    T, D = x.shape
    E = W.shape[1]
    tb = _TB
    steps = size // tb
    base = start // tb
    return pl.pallas_call(
        _affinity_block,
        grid=(steps,),
        in_specs=[
            pl.BlockSpec((tb, D), lambda i, b=base: (b + i, 0)),
            pl.BlockSpec((D, E), lambda i: (0, 0)),
        ],
        out_specs=pl.BlockSpec((tb, E), lambda i: (i, 0)),
        out_shape=jax.ShapeDtypeStruct((size, E), jnp.float32),
    )(x, W)


def _router_block(x_ref, visf_ref, w_ref, aux_ref, mb_ref,
                  idx_ref, gate_ref, aff_ref, *, n_experts, top_k):
    x = x_ref[...]
    logits = jnp.dot(x, w_ref[...], preferred_element_type=jnp.float32)
    sp = jnp.maximum(logits, 0.0) + jnp.log1p(jnp.exp(-jnp.abs(logits)))
    aff = jnp.sqrt(sp + 1e-12)

    visf = visf_ref[...]
    mb0 = mb_ref[0:1, :]
    mb1 = mb_ref[1:2, :]
    mrow = jnp.where(visf > 0.5, mb1, mb0)
    biased = aff + aux_ref[...] + mrow

    tb = biased.shape[0]
    iota = lax.broadcasted_iota(jnp.int32, (tb, n_experts), 1)
    neg_inf = jnp.float32(-jnp.inf)

    idx_cols = []
    gate_cols = []
    work = biased
    for _ in range(top_k):
        m = jnp.max(work, axis=-1, keepdims=True)
        cand = jnp.where(work == m, iota, n_experts)
        sel = jnp.min(cand, axis=-1, keepdims=True)
        pick = iota == sel
        g = jnp.sum(jnp.where(pick, aff, 0.0), axis=-1, keepdims=True)
        idx_cols.append(sel)
        gate_cols.append(g)
        work = jnp.where(pick, neg_inf, work)

    idx = jnp.concatenate(idx_cols, axis=1)
    gate_raw = jnp.concatenate(gate_cols, axis=1)
    gate = gate_raw / (jnp.sum(gate_raw, axis=-1, keepdims=True) + 1e-12)

    idx_ref[...] = idx
    gate_ref[...] = gate
    aff_ref[...] = aff


def _fused_router_call(x, visf, W, aux2, mb, start, size):
    T, D = x.shape
    E = W.shape[1]
    top_k = 8
    tb = _TBF
    steps = size // tb
    base = start // tb
    body = functools.partial(_router_block, n_experts=E, top_k=top_k)
    return pl.pallas_call(
        body,
        grid=(steps,),
        in_specs=[
            pl.BlockSpec((tb, D), lambda i, b=base: (b + i, 0)),
            pl.BlockSpec((tb, 1), lambda i, b=base: (b + i, 0)),
            pl.BlockSpec((D, E), lambda i: (0, 0)),
            pl.BlockSpec((1, E), lambda i: (0, 0)),
            pl.BlockSpec((2, E), lambda i: (0, 0)),
        ],
        out_specs=[
            pl.BlockSpec((tb, top_k), lambda i: (i, 0)),
            pl.BlockSpec((tb, top_k), lambda i: (i, 0)),
            pl.BlockSpec((tb, E), lambda i: (i, 0)),
        ],
        out_shape=[
            jax.ShapeDtypeStruct((size, top_k), jnp.int32),
            jax.ShapeDtypeStruct((size, top_k), jnp.float32),
            jax.ShapeDtypeStruct((size, E), jnp.float32),
        ],
    )(x, visf, W, aux2, mb)


def _sc_topk_body(aff_hbm, vis_hbm, aux_hbm, mb_hbm, idx_hbm, gate_hbm,
                  aff_v, vis_v, aux_v, mb_v, idxo_v, gateo_v, *, n_tok):
    wid = lax.axis_index("s") * 2 + lax.axis_index("c")
    base = wid * n_tok

    pltpu.sync_copy(aff_hbm.at[pl.ds(base, n_tok), :], aff_v)
    pltpu.sync_copy(vis_hbm.at[pl.ds(base, n_tok)], vis_v)
    pltpu.sync_copy(aux_hbm, aux_v)
    pltpu.sync_copy(mb_hbm, mb_v)

    ii = lax.broadcasted_iota(jnp.int32, (16,), 0)
    lo8 = ii < 8
    shifted = jnp.maximum(ii - 8, 0)

    pre = []
    for c in range(4):
        a_c = aux_v[pl.ds(c * 16, 16)]
        pre.append((a_c + mb_v[0, pl.ds(c * 16, 16)],
                    a_c + mb_v[1, pl.ds(c * 16, 16)]))

    def merge_top8(ak, av, bk, bv):
        ck = jnp.where(lo8, ak, bk.at[shifted].get(mode="promise_in_bounds"))
        cv = jnp.where(lo8, av, bv.at[shifted].get(mode="promise_in_bounds"))
        return plsc.sort_key_val(ck, cv, descending=True)

    def one_token(t):
        t_splat = jnp.full((16,), t, jnp.int32)
        vis_t = plsc.load_gather(vis_v, [t_splat])
        visb = vis_t != 0
        sk, sv = [], []
        for c in range(4):
            b_c = aff_v[t, pl.ds(c * 16, 16)] + jnp.where(visb, pre[c][1], pre[c][0])
            kk, vv = plsc.sort_key_val(b_c, ii + 16 * c, descending=True)
            sk.append(kk)
            sv.append(vv)
        mk0, mv0 = merge_top8(sk[0], sv[0], sk[1], sv[1])
        mk1, mv1 = merge_top8(sk[2], sv[2], sk[3], sv[3])
        _, fv = merge_top8(mk0, mv0, mk1, mv1)

        g_aff = plsc.load_gather(aff_v, [t_splat, fv], mask=lo8)
        g_aff = jnp.where(lo8, g_aff, 0.0)
        gate8 = g_aff / (jnp.sum(g_aff, axis=0) + 1e-12)
        return fv, gate8

    def tok_body(i, _):
        t = 2 * i
        fva, ga = one_token(t)
        fvb, gb = one_token(t + 1)
        idx16 = jnp.where(lo8, fva, fvb.at[shifted].get(mode="promise_in_bounds"))
        gate16 = jnp.where(lo8, ga, gb.at[shifted].get(mode="promise_in_bounds"))
        idxo_v[pl.ds(t * 8, 16)] = idx16
        gateo_v[pl.ds(t * 8, 16)] = gate16
        return _

    lax.fori_loop(0, n_tok // 2, tok_body, 0)

    pltpu.sync_copy(idxo_v, idx_hbm.at[pl.ds(base * 8, n_tok * 8)])
    pltpu.sync_copy(gateo_v, gate_hbm.at[pl.ds(base * 8, n_tok * 8)])


def _sc_topk_call(aff, visi, aux, mb):
    Tc, E = aff.shape
    n_tok = Tc // 32
    body = functools.partial(_sc_topk_body, n_tok=n_tok)
    fn = pl.kernel(
        body,
        out_type=[
            jax.ShapeDtypeStruct((Tc * 8,), jnp.int32),
            jax.ShapeDtypeStruct((Tc * 8,), jnp.float32),
        ],
        mesh=plsc.VectorSubcoreMesh(core_axis_name="c", subcore_axis_name="s"),
        compiler_params=pltpu.CompilerParams(needs_layout_passes=False),
        scratch_types=[
            pltpu.VMEM((n_tok, E), jnp.float32),
            pltpu.VMEM((n_tok,), jnp.int32),
            pltpu.VMEM((E,), jnp.float32),
            pltpu.VMEM((2, E), jnp.float32),
            pltpu.VMEM((n_tok * 8,), jnp.int32),
            pltpu.VMEM((n_tok * 8,), jnp.float32),
        ],
    )
    return fn(aff, visi, aux, mb)


def kernel(x, is_visual, W, aux_free_bias, modality_bias):
    T, D = x.shape
    E = W.shape[1]
    visi = is_visual.astype(jnp.int32)
    visf = is_visual.astype(jnp.float32).reshape(T, 1)
    aux2 = aux_free_bias.reshape(1, E)

    affs, idxs, gates = [], [], []
    start = 0
    for tc in _SC_CHUNKS:
        aff_c = _affinity_call(x, W, start, tc)
        idxf, gatef = _sc_topk_call(aff_c, visi[start:start + tc],
                                    aux_free_bias, modality_bias)
        affs.append(aff_c)
        idxs.append(idxf.reshape(tc, 8))
        gates.append(gatef.reshape(tc, 8))
        start += tc

    idx_t, gate_t, aff_t = _fused_router_call(
        x, visf, W, aux2, modality_bias, start, _TC_TAIL)
    idxs.append(idx_t)
    gates.append(gate_t)
    affs.append(aff_t)

    return (jnp.concatenate(idxs, axis=0),
            jnp.concatenate(gates, axis=0),
            jnp.concatenate(affs, axis=0))


# speedup vs baseline: 1.0827x; 1.0827x over previous
import functools

import jax
import jax.numpy as jnp
from jax import lax
from jax.experimental import pallas as pl
from jax.experimental.pallas import tpu as pltpu
from jax.experimental.pallas import tpu_sc as plsc

_TB = 1024
_TBF = 512
_SC_CHUNKS = (8192, 8192, 8192, 8192)


def _affinity_block(x_ref, w_ref, aff_ref):
    x = x_ref[...]
    logits = jnp.dot(x, w_ref[...], preferred_element_type=jnp.float32)
    sp = jnp.maximum(logits, 0.0) + jnp.log1p(jnp.exp(-jnp.abs(logits)))
    aff_ref[...] = jnp.sqrt(sp + 1e-12)


def _affinity_call(x, W, start, size):
    T, D = x.shape
    E = W.shape[1]
    tb = _TB
    steps = size // tb
    base = start // tb
    return pl.pallas_call(
        _affinity_block,
        grid=(steps,),
        in_specs=[
            pl.BlockSpec((tb, D), lambda i, b=base: (b + i, 0)),
            pl.BlockSpec((D, E), lambda i: (0, 0)),
        ],
        out_specs=pl.BlockSpec((tb, E), lambda i: (i, 0)),
        out_shape=jax.ShapeDtypeStruct((size, E), jnp.float32),
        compiler_params=pltpu.CompilerParams(dimension_semantics=('parallel',)),
    )(x, W)


def _router_block(x_ref, visf_ref, w_ref, aux_ref, mb_ref,
                  idx_ref, gate_ref, aff_ref, *, n_experts, top_k):
    x = x_ref[...]
    logits = jnp.dot(x, w_ref[...], preferred_element_type=jnp.float32)
    sp = jnp.maximum(logits, 0.0) + jnp.log1p(jnp.exp(-jnp.abs(logits)))
    aff = jnp.sqrt(sp + 1e-12)

    visf = visf_ref[...]
    mb0 = mb_ref[0:1, :]
    mb1 = mb_ref[1:2, :]
    mrow = jnp.where(visf > 0.5, mb1, mb0)
    biased = aff + aux_ref[...] + mrow

    tb = biased.shape[0]
    iota = lax.broadcasted_iota(jnp.int32, (tb, n_experts), 1)
    neg_inf = jnp.float32(-jnp.inf)

    idx_cols = []
    gate_cols = []
    work = biased
    for _ in range(top_k):
        m = jnp.max(work, axis=-1, keepdims=True)
        cand = jnp.where(work == m, iota, n_experts)
        sel = jnp.min(cand, axis=-1, keepdims=True)
        pick = iota == sel
        g = jnp.sum(jnp.where(pick, aff, 0.0), axis=-1, keepdims=True)
        idx_cols.append(sel)
        gate_cols.append(g)
        work = jnp.where(pick, neg_inf, work)

    idx = jnp.concatenate(idx_cols, axis=1)
    gate_raw = jnp.concatenate(gate_cols, axis=1)
    gate = gate_raw / (jnp.sum(gate_raw, axis=-1, keepdims=True) + 1e-12)

    idx_ref[...] = idx
    gate_ref[...] = gate
    aff_ref[...] = aff


def _fused_router_call(x, visf, W, aux2, mb, start, size):
    T, D = x.shape
    E = W.shape[1]
    top_k = 8
    tb = _TBF
    steps = size // tb
    base = start // tb
    body = functools.partial(_router_block, n_experts=E, top_k=top_k)
    return pl.pallas_call(
        body,
        grid=(steps,),
        in_specs=[
            pl.BlockSpec((tb, D), lambda i, b=base: (b + i, 0)),
            pl.BlockSpec((tb, 1), lambda i, b=base: (b + i, 0)),
            pl.BlockSpec((D, E), lambda i: (0, 0)),
            pl.BlockSpec((1, E), lambda i: (0, 0)),
            pl.BlockSpec((2, E), lambda i: (0, 0)),
        ],
        out_specs=[
            pl.BlockSpec((tb, top_k), lambda i: (i, 0)),
            pl.BlockSpec((tb, top_k), lambda i: (i, 0)),
            pl.BlockSpec((tb, E), lambda i: (i, 0)),
        ],
        out_shape=[
            jax.ShapeDtypeStruct((size, top_k), jnp.int32),
            jax.ShapeDtypeStruct((size, top_k), jnp.float32),
            jax.ShapeDtypeStruct((size, E), jnp.float32),
        ],
    )(x, visf, W, aux2, mb)


def _sc_topk_body(aff_hbm, vis_hbm, aux_hbm, mb_hbm, idx_hbm, gate_hbm,
                  aff_v, vis_v, aux_v, mb_v, idxo_v, gateo_v, *, n_tok):
    wid = lax.axis_index("s") * 2 + lax.axis_index("c")
    base = wid * n_tok

    pltpu.sync_copy(aff_hbm.at[pl.ds(base, n_tok), :], aff_v)
    pltpu.sync_copy(vis_hbm.at[pl.ds(base, n_tok)], vis_v)
    pltpu.sync_copy(aux_hbm, aux_v)
    pltpu.sync_copy(mb_hbm, mb_v)

    ii = lax.broadcasted_iota(jnp.int32, (16,), 0)
    lo8 = ii < 8
    shifted = jnp.maximum(ii - 8, 0)

    pre = []
    for c in range(4):
        a_c = aux_v[pl.ds(c * 16, 16)]
        pre.append((a_c + mb_v[0, pl.ds(c * 16, 16)],
                    a_c + mb_v[1, pl.ds(c * 16, 16)]))

    def merge_top8(ak, av, bk, bv):
        ck = jnp.where(lo8, ak, bk.at[shifted].get(mode="promise_in_bounds"))
        cv = jnp.where(lo8, av, bv.at[shifted].get(mode="promise_in_bounds"))
        return plsc.sort_key_val(ck, cv, descending=True)

    def one_token(t):
        t_splat = jnp.full((16,), t, jnp.int32)
        vis_t = plsc.load_gather(vis_v, [t_splat])
        visb = vis_t != 0
        sk, sv = [], []
        for c in range(4):
            b_c = aff_v[t, pl.ds(c * 16, 16)] + jnp.where(visb, pre[c][1], pre[c][0])
            kk, vv = plsc.sort_key_val(b_c, ii + 16 * c, descending=True)
            sk.append(kk)
            sv.append(vv)
        mk0, mv0 = merge_top8(sk[0], sv[0], sk[1], sv[1])
        mk1, mv1 = merge_top8(sk[2], sv[2], sk[3], sv[3])
        _, fv = merge_top8(mk0, mv0, mk1, mv1)

        g_aff = plsc.load_gather(aff_v, [t_splat, fv], mask=lo8)
        g_aff = jnp.where(lo8, g_aff, 0.0)
        gate8 = g_aff / (jnp.sum(g_aff, axis=0) + 1e-12)
        return fv, gate8

    def tok_body(i, _):
        t = 2 * i
        fva, ga = one_token(t)
        fvb, gb = one_token(t + 1)
        idx16 = jnp.where(lo8, fva, fvb.at[shifted].get(mode="promise_in_bounds"))
        gate16 = jnp.where(lo8, ga, gb.at[shifted].get(mode="promise_in_bounds"))
        idxo_v[pl.ds(t * 8, 16)] = idx16
        gateo_v[pl.ds(t * 8, 16)] = gate16
        return _

    lax.fori_loop(0, n_tok // 2, tok_body, 0)

    pltpu.sync_copy(idxo_v, idx_hbm.at[pl.ds(base * 8, n_tok * 8)])
    pltpu.sync_copy(gateo_v, gate_hbm.at[pl.ds(base * 8, n_tok * 8)])


def _sc_topk_call(aff, visi, aux, mb):
    Tc, E = aff.shape
    n_tok = Tc // 32
    body = functools.partial(_sc_topk_body, n_tok=n_tok)
    fn = pl.kernel(
        body,
        out_type=[
            jax.ShapeDtypeStruct((Tc * 8,), jnp.int32),
            jax.ShapeDtypeStruct((Tc * 8,), jnp.float32),
        ],
        mesh=plsc.VectorSubcoreMesh(core_axis_name="c", subcore_axis_name="s"),
        compiler_params=pltpu.CompilerParams(needs_layout_passes=False),
        scratch_types=[
            pltpu.VMEM((n_tok, E), jnp.float32),
            pltpu.VMEM((n_tok,), jnp.int32),
            pltpu.VMEM((E,), jnp.float32),
            pltpu.VMEM((2, E), jnp.float32),
            pltpu.VMEM((n_tok * 8,), jnp.int32),
            pltpu.VMEM((n_tok * 8,), jnp.float32),
        ],
    )
    return fn(aff, visi, aux, mb)


def kernel(x, is_visual, W, aux_free_bias, modality_bias):
    T, D = x.shape
    E = W.shape[1]
    visi = is_visual.astype(jnp.int32)
    visf = is_visual.astype(jnp.float32).reshape(T, 1)
    aux2 = aux_free_bias.reshape(1, E)

    affs, idxs, gates = [], [], []
    start = 0
    for tc in _SC_CHUNKS:
        aff_c = _affinity_call(x, W, start, tc)
        idxf, gatef = _sc_topk_call(aff_c, visi[start:start + tc],
                                    aux_free_bias, modality_bias)
        affs.append(aff_c)
        idxs.append(idxf.reshape(tc, 8))
        gates.append(gatef.reshape(tc, 8))
        start += tc

    return (jnp.concatenate(idxs, axis=0),
            jnp.concatenate(gates, axis=0),
            jnp.concatenate(affs, axis=0))
